# Initial kernel scaffold; baseline (speedup 1.0000x reference)
import jax
import jax.numpy as jnp
from jax.experimental import pallas as pl


def kernel(x, edge_index, edge_attr, mlp1_params, mlp2_params, dec_params, mode):
    raise NotImplementedError("write your pallas kernel here")


# speedup vs baseline: 3.6838x; 3.6838x over previous
import functools

import jax
import jax.numpy as jnp
from jax import lax
from jax.experimental import pallas as pl
from jax.experimental.pallas import tpu as pltpu
from jax.experimental.pallas import tpu_sc as plsc

_CH = 128


def _sc_gather(x, row, col):
  n, feat = x.shape
  e = row.shape[0]
  info = plsc.get_sparse_core_info()
  nc, ns = info.num_cores, info.num_subcores
  nw = nc * ns
  n_chunks = e // _CH
  iters = (n_chunks + nw - 1) // nw

  mesh = plsc.VectorSubcoreMesh(core_axis_name="c", subcore_axis_name="s")

  @functools.partial(
      pl.kernel,
      mesh=mesh,
      out_type=(jax.ShapeDtypeStruct((e, feat), jnp.float32),
                jax.ShapeDtypeStruct((e, feat), jnp.float32)),
      scratch_types=[
          pltpu.VMEM((_CH,), jnp.int32),
          pltpu.VMEM((_CH,), jnp.int32),
          pltpu.VMEM((_CH, feat), jnp.float32),
          pltpu.VMEM((_CH, feat), jnp.float32),
          pltpu.SemaphoreType.DMA,
          pltpu.SemaphoreType.DMA,
      ],
      compiler_params=pltpu.CompilerParams(use_tc_tiling_on_sc=False),
  )
  def k(x_hbm, row_hbm, col_hbm, src_out, dst_out,
        idx_r, idx_c, rows_r, rows_c, sem_r, sem_c):
    wid = lax.axis_index("s") * nc + lax.axis_index("c")

    def body(i, carry):
      chunk = wid + i * nw

      @pl.when(chunk < n_chunks)
      def _():
        base = chunk * _CH
        pltpu.sync_copy(row_hbm.at[pl.ds(base, _CH)], idx_r)
        pltpu.sync_copy(col_hbm.at[pl.ds(base, _CH)], idx_c)
        cp_r = pltpu.async_copy(x_hbm.at[idx_r], rows_r, sem_r)
        cp_c = pltpu.async_copy(x_hbm.at[idx_c], rows_c, sem_c)
        cp_r.wait()
        cp_c.wait()
        pltpu.sync_copy(rows_r, src_out.at[pl.ds(base, _CH)])
        pltpu.sync_copy(rows_c, dst_out.at[pl.ds(base, _CH)])

      return carry

    lax.fori_loop(0, iters, body, 0)

  return k(x, row, col)


def _tc_edge_mlp(src, dst, ea, mlp1_params):
  (w1, b1), (w2, b2), (w3, b3) = mlp1_params
  e = src.shape[0]
  h = w1.shape[1]
  blk = 4000
  grid = e // blk

  w1p = jnp.concatenate([w1, jnp.zeros((6, h), jnp.float32)], axis=0)
  b1p = b1.reshape(1, h)
  b2p = b2.reshape(1, h)
  w3p = jnp.concatenate([w3, jnp.zeros((h, 4), jnp.float32)], axis=1)
  b3p = jnp.concatenate(
      [b3, jnp.array([1.0, 0.0, 0.0, 0.0], jnp.float32)]).reshape(1, 8)

  def body(src_ref, dst_ref, ea_ref, w1_ref, b1_ref, w2_ref, b2_ref,
           w3_ref, b3_ref, out_ref):
    s = src_ref[...]
    d = dst_ref[...]
    att = ea_ref[...]
    disp = d[:, 0:3] - s[:, 0:3]
    nrm = jnp.sqrt(jnp.sum(disp * disp, axis=1, keepdims=True) + 1e-12)
    ni = jnp.concatenate(
        [disp, nrm, att, s[:, 15:16], d[:, 15:16],
         jnp.zeros((blk, 6), jnp.float32)], axis=1)
    hh = jnp.maximum(
        jnp.dot(ni, w1_ref[...], preferred_element_type=jnp.float32)
        + b1_ref[...], 0.0)
    hh = jnp.maximum(
        jnp.dot(hh, w2_ref[...], preferred_element_type=jnp.float32)
        + b2_ref[...], 0.0)
    oo = (jnp.dot(hh, w3_ref[...], preferred_element_type=jnp.float32)
          + b3_ref[...])
    out_ref[...] = oo + jnp.concatenate(
        [att, jnp.zeros((blk, 4), jnp.float32)], axis=1)

  wspec = lambda shape: pl.BlockSpec(shape, lambda i: (0, 0))
  return pl.pallas_call(
      body,
      grid=(grid,),
      in_specs=[
          pl.BlockSpec((blk, 16), lambda i: (i, 0)),
          pl.BlockSpec((blk, 16), lambda i: (i, 0)),
          pl.BlockSpec((blk, 4), lambda i: (i, 0)),
          wspec((16, h)), wspec((1, h)),
          wspec((h, h)), wspec((1, h)),
          wspec((h, 8)), wspec((1, 8)),
      ],
      out_specs=pl.BlockSpec((blk, 8), lambda i: (i, 0)),
      out_shape=jax.ShapeDtypeStruct((e, 8), jnp.float32),
  )(src, dst, ea, w1p, b1p, w2, b2p, w3p, b3p)


def _sc_scatter(e8, col, n_pad):
  e = e8.shape[0]
  info = plsc.get_sparse_core_info()
  nc, ns = info.num_cores, info.num_subcores
  nw = nc * ns
  n_chunks = e // _CH
  iters = (n_chunks + nw - 1) // nw
  rows_per_tile = n_pad // ns

  zeros8 = jnp.zeros((n_pad, 8), jnp.float32)
  mesh = plsc.VectorSubcoreMesh(core_axis_name="c", subcore_axis_name="s")

  @functools.partial(
      pl.kernel,
      mesh=mesh,
      out_type=jax.ShapeDtypeStruct((nc, n_pad, 8), jnp.float32),
      scratch_types=[
          pltpu.VMEM((_CH,), jnp.int32),
          pltpu.VMEM((_CH, 8), jnp.float32),
          pltpu.VMEM_SHARED((n_pad, 8), jnp.float32),
      ],
      compiler_params=pltpu.CompilerParams(use_tc_tiling_on_sc=False),
  )
  def k(e_hbm, col_hbm, z_hbm, out_hbm, idx_v, ev, acc):
    cid = lax.axis_index("c")
    sid = lax.axis_index("s")
    wid = sid * nc + cid
    r0 = sid * rows_per_tile

    pltpu.sync_copy(z_hbm.at[pl.ds(r0, rows_per_tile)],
                    acc.at[pl.ds(r0, rows_per_tile)])
    plsc.subcore_barrier()

    def body(i, carry):
      chunk = wid + i * nw

      @pl.when(chunk < n_chunks)
      def _():
        base = chunk * _CH
        pltpu.sync_copy(col_hbm.at[pl.ds(base, _CH)], idx_v)
        pltpu.sync_copy(e_hbm.at[pl.ds(base, _CH)], ev)
        pltpu.sync_copy(ev, acc.at[idx_v], add=True)

      return carry

    lax.fori_loop(0, iters, body, 0)
    plsc.subcore_barrier()

    pltpu.sync_copy(acc.at[pl.ds(r0, rows_per_tile)],
                    out_hbm.at[cid].at[pl.ds(r0, rows_per_tile)])

  return k(e8, col, zeros8)


def _tc_node_dec(x, p0, p1, mlp2_params, dec_params, mode):
  (w21, b21), (w22, b22), (w23, b23) = mlp2_params
  n = x.shape[0]
  h = w21.shape[1]
  t = dec_params[-1][0].shape[1]
  blk = 2000
  grid = n // blk

  w23p = jnp.concatenate([w23, jnp.zeros((h, 7), jnp.float32)], axis=1)
  b23p = jnp.concatenate([b23, jnp.zeros((7,), jnp.float32)]).reshape(1, 8)
  dec_flat = []
  for (wd, bd) in dec_params:
    dec_flat.append(wd)
    dec_flat.append(bd.reshape(1, -1))
  mode_arr = jnp.reshape(jnp.asarray(mode, jnp.int32), (1, 1))

  def body(x_ref, p0_ref, p1_ref, w21_ref, b21_ref, w22_ref, b22_ref,
           w23_ref, b23_ref, d1_ref, db1_ref, d2_ref, db2_ref, d3_ref,
           db3_ref, d4_ref, db4_ref, mode_ref, out_ref):
    xx = x_ref[...]
    ps = p0_ref[...] + p1_ref[...]
    cnt = jnp.maximum(ps[:, 4:5], 1.0)
    aggr = ps[:, 0:4] / cnt
    ni = jnp.concatenate([xx[:, 14:16], aggr], axis=1)
    hh = jnp.maximum(
        jnp.dot(ni, w21_ref[...], preferred_element_type=jnp.float32)
        + b21_ref[...], 0.0)
    hh = jnp.maximum(
        jnp.dot(hh, w22_ref[...], preferred_element_type=jnp.float32)
        + b22_ref[...], 0.0)
    delta = (jnp.dot(hh, w23_ref[...], preferred_element_type=jnp.float32)
             + b23_ref[...])[:, 0:1]
    lastcol = (lax.broadcasted_iota(jnp.int32, (1, 16), 1) == 15)
    x_res = xx + delta * lastcol.astype(jnp.float32)
    x_new = xx + jnp.maximum(x_res, 0.0)
    hh = jnp.maximum(
        jnp.dot(x_new, d1_ref[...], preferred_element_type=jnp.float32)
        + db1_ref[...], 0.0)
    hh = jnp.maximum(
        jnp.dot(hh, d2_ref[...], preferred_element_type=jnp.float32)
        + db2_ref[...], 0.0)
    hh = jnp.maximum(
        jnp.dot(hh, d3_ref[...], preferred_element_type=jnp.float32)
        + db3_ref[...], 0.0)
    oo = (jnp.dot(hh, d4_ref[...], preferred_element_type=jnp.float32)
          + db4_ref[...])
    mask = (mode_ref[0, 0] == 1).astype(jnp.float32)
    out_ref[...] = oo * mask

  wspec = lambda shape: pl.BlockSpec(shape, lambda i: (0, 0))
  return pl.pallas_call(
      body,
      grid=(grid,),
      in_specs=[
          pl.BlockSpec((blk, 16), lambda i: (i, 0)),
          pl.BlockSpec((blk, 8), lambda i: (i, 0)),
          pl.BlockSpec((blk, 8), lambda i: (i, 0)),
          wspec((6, h)), wspec((1, h)),
          wspec((h, h)), wspec((1, h)),
          wspec((h, 8)), wspec((1, 8)),
          wspec((16, h)), wspec((1, h)),
          wspec((h, h)), wspec((1, h)),
          wspec((h, h)), wspec((1, h)),
          wspec((h, t)), wspec((1, t)),
          pl.BlockSpec(memory_space=pltpu.SMEM),
      ],
      out_specs=pl.BlockSpec((blk, t), lambda i: (i, 0)),
      out_shape=jax.ShapeDtypeStruct((n, t), jnp.float32),
  )(x, p0, p1, w21, b21.reshape(1, h), w22, b22.reshape(1, h), w23p, b23p,
    *dec_flat, mode_arr)


def kernel(x, edge_index, edge_attr, mlp1_params, mlp2_params, dec_params,
           mode):
  n = x.shape[0]
  n_pad = ((n + _CH - 1) // _CH) * _CH
  row = edge_index[0]
  col = edge_index[1]
  src_rows, dst_rows = _sc_gather(x, row, col)
  e8 = _tc_edge_mlp(src_rows, dst_rows, edge_attr, mlp1_params)
  parts = _sc_scatter(e8, col, n_pad)
  p0 = parts[0, :n]
  p1 = parts[1, :n]
  return _tc_node_dec(x, p0, p1, mlp2_params, dec_params, mode)
